# baseline (speedup 1.0000x reference)
import functools

import jax
import jax.numpy as jnp
from jax import lax
from jax.experimental import pallas as pl
from jax.experimental.pallas import tpu as pltpu
from jax.experimental.pallas import tpu_sc as plsc

B = 128
C = 64
DIM = 64
NUM_CANDS = 100000
TOP_K = 10

SC_CORES = 2
SC_SUBCORES = 16
NW = SC_CORES * SC_SUBCORES

def _vmesh():
    return plsc.VectorSubcoreMesh(core_axis_name="c", subcore_axis_name="s",
                                  num_cores=SC_CORES,
                                  num_subcores=SC_SUBCORES)


_N_CAND_IDX = B * C
_CAND_PER_W = _N_CAND_IDX // NW
_WORD_PER_W = (2 * B) // NW


def _row_gather(tab_hbm, idx_v, out_hbm, rows_v, sem, base, n):
    for g in range(n // 16):
        iv = idx_v[pl.ds(g * 16, 16)]
        for k in range(16):
            j = g * 16 + k
            pltpu.make_async_copy(tab_hbm.at[iv[k]], rows_v.at[j],
                                  sem).start()

    def _drain(j, carry):
        pltpu.make_async_copy(tab_hbm.at[0], rows_v.at[0], sem).wait()
        return carry
    lax.fori_loop(0, n, _drain, 0)
    pltpu.sync_copy(rows_v, out_hbm.at[pl.ds(base, n)])


def _sc_gather_body(widx_hbm, cl_hbm, cr_hbm, wtab_hbm, ctab_hbm,
                    wout, clout, crout, widx_v, wrows_v, idx_v,
                    rows_v, sem):
    wid = lax.axis_index("s") * SC_CORES + lax.axis_index("c")
    wbase = wid * _WORD_PER_W
    pltpu.sync_copy(widx_hbm.at[pl.ds(wbase, _WORD_PER_W)],
                    widx_v.at[pl.ds(0, _WORD_PER_W)])
    iv = widx_v[...]
    for k in range(_WORD_PER_W):
        pltpu.make_async_copy(wtab_hbm.at[iv[k]], wrows_v.at[k], sem).start()
    for k in range(_WORD_PER_W):
        pltpu.make_async_copy(wtab_hbm.at[0], wrows_v.at[0], sem).wait()
    pltpu.sync_copy(wrows_v, wout.at[pl.ds(wbase, _WORD_PER_W)])

    cbase = wid * _CAND_PER_W
    pltpu.sync_copy(cl_hbm.at[pl.ds(cbase, _CAND_PER_W)], idx_v)
    _row_gather(ctab_hbm, idx_v, clout, rows_v, sem, cbase, _CAND_PER_W)

    pltpu.sync_copy(cr_hbm.at[pl.ds(cbase, _CAND_PER_W)], idx_v)
    _row_gather(ctab_hbm, idx_v, crout, rows_v, sem, cbase, _CAND_PER_W)


@functools.lru_cache(maxsize=None)
def _sc_gather_kernel():
    return pl.kernel(
        _sc_gather_body,
        mesh=_vmesh(),
        out_type=[
            jax.ShapeDtypeStruct((2 * B, DIM), jnp.float32),
            jax.ShapeDtypeStruct((_N_CAND_IDX, DIM), jnp.float32),
            jax.ShapeDtypeStruct((_N_CAND_IDX, DIM), jnp.float32),
        ],
        scratch_types=[
            pltpu.VMEM((16,), jnp.int32),
            pltpu.VMEM((_WORD_PER_W, DIM), jnp.float32),
            pltpu.VMEM((_CAND_PER_W,), jnp.int32),
            pltpu.VMEM((_CAND_PER_W, DIM), jnp.float32),
            pltpu.SemaphoreType.DMA,
        ],
        compiler_params=pltpu.CompilerParams(needs_layout_passes=False),
    )


_TW = 2560
_NT = pl.cdiv(NUM_CANDS, _TW)


def _attn_body(wl_ref, wr_ref, ecl_ref, eclT_ref, ecr_ref, ecrT_ref,
               Wa_ref, ba_ref, Wc1_ref, Wc2_ref, bc_ref, pv_ref):
    wl = wl_ref[...]
    wr = wr_ref[...]

    def bf(x):
        return x.astype(jnp.bfloat16).astype(jnp.float32)

    wa_l = jnp.tanh(wl @ Wa_ref[...] + ba_ref[...])
    wa_r = jnp.tanh(wr @ Wa_ref[...] + ba_ref[...])

    sc_l = jnp.sum(bf(ecr_ref[...]) * bf(wa_l)[:, None, :], axis=2)
    sc_l = sc_l - jnp.max(sc_l, axis=1, keepdims=True)
    a_l = jnp.exp(sc_l)
    a_l = a_l / jnp.sum(a_l, axis=1, keepdims=True)
    agg_l = jnp.sum(ecrT_ref[...] * a_l[:, None, :], axis=2)

    sc_r = jnp.sum(bf(ecl_ref[...]) * bf(wa_r)[:, None, :], axis=2)
    sc_r = sc_r - jnp.max(sc_r, axis=1, keepdims=True)
    a_r = jnp.exp(sc_r)
    a_r = a_r / jnp.sum(a_r, axis=1, keepdims=True)
    agg_r = jnp.sum(eclT_ref[...] * a_r[:, None, :], axis=2)

    ww = wl + wr
    cw = agg_l + agg_r
    pv_ref[...] = jnp.tanh(ww @ Wc1_ref[...] + cw @ Wc2_ref[...]
                           + bc_ref[...])


def _attention(wl, wr, ecl3, ecl3T, ecr3, ecr3T, W_a, b_a, Wc1, Wc2, b_c):
    return pl.pallas_call(
        _attn_body,
        out_shape=jax.ShapeDtypeStruct((B, DIM), jnp.float32),
    )(wl, wr, ecl3, ecl3T, ecr3, ecr3T, W_a, b_a, Wc1, Wc2, b_c)


def _tc_body(pv_in_ref, tab_ref, out_ref, lg_ref, m_ref, s_ref):
    ph = pl.program_id(0)
    t = pl.program_id(1)

    @pl.when((ph == 0) & (t == 0))
    def _init():
        m_ref[...] = jnp.full((B, 1), -jnp.inf, jnp.float32)
        s_ref[...] = jnp.zeros((B, 1), jnp.float32)

    @pl.when(ph == 0)
    def _phase_stats():
        logits = lax.dot_general(pv_in_ref[...], tab_ref[...],
                                 (((1,), (1,)), ((), ())),
                                 preferred_element_type=jnp.float32)
        @pl.when(t == _NT - 1)
        def _mask():
            col = t * _TW + lax.broadcasted_iota(jnp.int32, (B, _TW), 1)
            lg_ref[:, pl.ds(t * _TW, _TW)] = jnp.where(
                col < NUM_CANDS, logits, -jnp.inf)
        @pl.when(t != _NT - 1)
        def _nomask():
            lg_ref[:, pl.ds(t * _TW, _TW)] = logits

        lg = lg_ref[:, pl.ds(t * _TW, _TW)]
        tmax = jnp.max(lg, axis=1, keepdims=True)
        m_old = m_ref[...]
        m_new = jnp.maximum(m_old, tmax)
        s_ref[...] = (s_ref[...] * jnp.exp(m_old - m_new)
                      + jnp.sum(jnp.exp(lg - m_new), axis=1, keepdims=True))
        m_ref[...] = m_new

    @pl.when(ph == 1)
    def _phase_write():
        lg = lg_ref[:, pl.ds(t * _TW, _TW)]
        out_ref[...] = jnp.exp(lg - m_ref[...]) / s_ref[...]


def _tc_softmax(pv, cand_tab):
    return pl.pallas_call(
        _tc_body,
        grid=(2, _NT),
        in_specs=[
            pl.BlockSpec((B, DIM), lambda ph, t: (0, 0)),
            pl.BlockSpec((_TW, DIM),
                         lambda ph, t: (jnp.where(ph == 0, t, 0), 0)),
        ],
        out_specs=pl.BlockSpec((B, _TW),
                               lambda ph, t: (0, jnp.where(ph == 0, 0, t))),
        out_shape=jax.ShapeDtypeStruct((B, NUM_CANDS), jnp.float32),
        scratch_shapes=[
            pltpu.VMEM((B, _NT * _TW), jnp.float32),
            pltpu.VMEM((B, 1), jnp.float32),
            pltpu.VMEM((B, 1), jnp.float32),
        ],
        compiler_params=pltpu.CompilerParams(
            dimension_semantics=("arbitrary", "arbitrary")),
    )(pv, cand_tab)


_ROWS_PER_W = B // NW
_BLK = 800
_NBLK = NUM_CANDS // _BLK
_VPB = _BLK // 16


def _sc_topk_body(y_hbm, out_hbm, buf, bmax, idxv):
    wid = lax.axis_index("s") * SC_CORES + lax.axis_index("c")
    neg = jnp.full((16,), -1.0, jnp.float32)
    lane = lax.iota(jnp.int32, 16)
    lane0 = lane == 0

    for r in range(_ROWS_PER_W):
        row = wid * _ROWS_PER_W + r
        pltpu.sync_copy(y_hbm.at[row], buf)
        idxv[...] = jnp.zeros((16,), jnp.int32)
        for i in range(8):
            bmax[pl.ds(i * 16, 16)] = neg

        def _bm(j, carry):
            accs = [neg, neg, neg, neg]
            for i in range(_VPB):
                accs[i % 4] = jnp.maximum(
                    accs[i % 4], buf[pl.ds(j * _BLK + i * 16, 16)])
            mx = jnp.max(jnp.maximum(jnp.maximum(accs[0], accs[1]),
                                     jnp.maximum(accs[2], accs[3])))
            plsc.store_scatter(bmax, [jnp.broadcast_to(j, (16,))],
                               jnp.broadcast_to(mx, (16,)), mask=lane0)
            return carry
        lax.fori_loop(0, _NBLK, _bm, 0)

        def _sel(k, carry):
            acc = neg
            for i in range(8):
                acc = jnp.maximum(acc, bmax[pl.ds(i * 16, 16)])
            mx = jnp.max(acc)
            mxs = jnp.broadcast_to(mx, (16,))

            jbv = jnp.full((16,), _NBLK + 16, jnp.int32)
            for i in range(8):
                v = bmax[pl.ds(i * 16, 16)]
                jbv = jnp.minimum(jbv, jnp.where(v == mxs, lane + i * 16,
                                                 _NBLK + 16))
            jb = jnp.min(jbv)
            off = jb * _BLK

            fidv = jnp.full((16,), NUM_CANDS + 16, jnp.int32)
            for i in range(_VPB):
                v = buf[pl.ds(off + i * 16, 16)]
                fidv = jnp.minimum(fidv,
                                   jnp.where(v == mxs,
                                             lane + (off + i * 16),
                                             jnp.int32(NUM_CANDS + 16)))
            fid = jnp.min(fidv)

            plsc.store_scatter(idxv, [jnp.broadcast_to(k, (16,))],
                               jnp.broadcast_to(fid, (16,)), mask=lane0)
            plsc.store_scatter(buf, [jnp.broadcast_to(fid, (16,))],
                               neg, mask=lane0)

            naccs = [neg, neg, neg, neg]
            for i in range(_VPB):
                naccs[i % 4] = jnp.maximum(
                    naccs[i % 4], buf[pl.ds(off + i * 16, 16)])
            nbm = jnp.max(jnp.maximum(jnp.maximum(naccs[0], naccs[1]),
                                      jnp.maximum(naccs[2], naccs[3])))
            plsc.store_scatter(bmax, [jnp.broadcast_to(jb, (16,))],
                               jnp.broadcast_to(nbm, (16,)), mask=lane0)
            return carry
        lax.fori_loop(0, TOP_K, _sel, 0)

        pltpu.sync_copy(idxv, out_hbm.at[row])


@functools.lru_cache(maxsize=None)
def _sc_topk_kernel():
    return pl.kernel(
        _sc_topk_body,
        mesh=_vmesh(),
        out_type=jax.ShapeDtypeStruct((B, 16), jnp.int32),
        scratch_types=[
            pltpu.VMEM((NUM_CANDS,), jnp.float32),
            pltpu.VMEM((128,), jnp.float32),
            pltpu.VMEM((16,), jnp.int32),
        ],
        compiler_params=pltpu.CompilerParams(needs_layout_passes=False),
    )


def kernel(entities, candidates_l, candidates_r, word_embeds,
           candidate_embeds, W_a, b_a, W_c, b_c):
    widx = jnp.concatenate([entities[:, 2], entities[:, 3]])
    cl = candidates_l.reshape(-1)
    cr = candidates_r.reshape(-1)

    wrows, ecl, ecr = _sc_gather_kernel()(widx, cl, cr, word_embeds,
                                          candidate_embeds)
    wl, wr = wrows[:B], wrows[B:]
    ecl3 = ecl.reshape(B, C, DIM)
    ecr3 = ecr.reshape(B, C, DIM)
    ecl3T = ecl3.swapaxes(1, 2)
    ecr3T = ecr3.swapaxes(1, 2)

    pv = _attention(wl, wr, ecl3, ecl3T, ecr3, ecr3T,
                    W_a, b_a, W_c[:DIM], W_c[DIM:], b_c)
    y_pred = _tc_softmax(pv, candidate_embeds)
    top16 = _sc_topk_kernel()(y_pred)
    return y_pred, top16[:, :TOP_K]


# speedup vs baseline: 1.0742x; 1.0742x over previous
import functools

import jax
import jax.numpy as jnp
from jax import lax
from jax.experimental import pallas as pl
from jax.experimental.pallas import tpu as pltpu
from jax.experimental.pallas import tpu_sc as plsc

B = 128
C = 64
DIM = 64
NUM_CANDS = 100000
TOP_K = 10

SC_CORES = 2
SC_SUBCORES = 16
NW = SC_CORES * SC_SUBCORES

def _vmesh():
    return plsc.VectorSubcoreMesh(core_axis_name="c", subcore_axis_name="s",
                                  num_cores=SC_CORES,
                                  num_subcores=SC_SUBCORES)


_N_CAND_IDX = B * C
_CAND_PER_W = _N_CAND_IDX // NW
_WORD_PER_W = (2 * B) // NW


def _row_gather(tab_hbm, idx_v, out_hbm, rows_v, sem, base, n):
    for g in range(n // 16):
        iv = idx_v[pl.ds(g * 16, 16)]
        for k in range(16):
            j = g * 16 + k
            pltpu.make_async_copy(tab_hbm.at[iv[k]], rows_v.at[j],
                                  sem).start()

    def _drain(j, carry):
        pltpu.make_async_copy(tab_hbm.at[0], rows_v.at[0], sem).wait()
        return carry
    lax.fori_loop(0, n, _drain, 0)
    pltpu.sync_copy(rows_v, out_hbm.at[pl.ds(base, n)])


def _sc_gather_body(widx_hbm, cl_hbm, cr_hbm, wtab_hbm, ctab_hbm,
                    wout, clout, crout, widx_v, wrows_v, idx_v,
                    rows_v, sem):
    wid = lax.axis_index("s") * SC_CORES + lax.axis_index("c")
    wbase = wid * _WORD_PER_W
    pltpu.sync_copy(widx_hbm.at[pl.ds(wbase, _WORD_PER_W)],
                    widx_v.at[pl.ds(0, _WORD_PER_W)])
    iv = widx_v[...]
    for k in range(_WORD_PER_W):
        pltpu.make_async_copy(wtab_hbm.at[iv[k]], wrows_v.at[k], sem).start()
    for k in range(_WORD_PER_W):
        pltpu.make_async_copy(wtab_hbm.at[0], wrows_v.at[0], sem).wait()
    pltpu.sync_copy(wrows_v, wout.at[pl.ds(wbase, _WORD_PER_W)])

    cbase = wid * _CAND_PER_W
    pltpu.sync_copy(cl_hbm.at[pl.ds(cbase, _CAND_PER_W)], idx_v)
    _row_gather(ctab_hbm, idx_v, clout, rows_v, sem, cbase, _CAND_PER_W)

    pltpu.sync_copy(cr_hbm.at[pl.ds(cbase, _CAND_PER_W)], idx_v)
    _row_gather(ctab_hbm, idx_v, crout, rows_v, sem, cbase, _CAND_PER_W)


@functools.lru_cache(maxsize=None)
def _sc_gather_kernel():
    return pl.kernel(
        _sc_gather_body,
        mesh=_vmesh(),
        out_type=[
            jax.ShapeDtypeStruct((2 * B, DIM), jnp.float32),
            jax.ShapeDtypeStruct((_N_CAND_IDX, DIM), jnp.float32),
            jax.ShapeDtypeStruct((_N_CAND_IDX, DIM), jnp.float32),
        ],
        scratch_types=[
            pltpu.VMEM((16,), jnp.int32),
            pltpu.VMEM((_WORD_PER_W, DIM), jnp.float32),
            pltpu.VMEM((_CAND_PER_W,), jnp.int32),
            pltpu.VMEM((_CAND_PER_W, DIM), jnp.float32),
            pltpu.SemaphoreType.DMA,
        ],
        compiler_params=pltpu.CompilerParams(needs_layout_passes=False),
    )


_TW = 2560
_NT = pl.cdiv(NUM_CANDS, _TW)


def _attn_body(wl_ref, wr_ref, ecl_ref, ecr_ref,
               Wa_ref, ba_ref, Wc1_ref, Wc2_ref, bc_ref, pv_ref):
    wl = wl_ref[...]
    wr = wr_ref[...]

    def bf(x):
        return x.astype(jnp.bfloat16).astype(jnp.float32)

    wa_l = jnp.tanh(wl @ Wa_ref[...] + ba_ref[...])
    wa_r = jnp.tanh(wr @ Wa_ref[...] + ba_ref[...])

    sc_l = jnp.sum(bf(ecr_ref[...]) * bf(wa_l)[:, None, :], axis=2)
    sc_l = sc_l - jnp.max(sc_l, axis=1, keepdims=True)
    a_l = jnp.exp(sc_l)
    a_l = a_l / jnp.sum(a_l, axis=1, keepdims=True)
    agg_l = jnp.sum(ecr_ref[...] * a_l[:, :, None], axis=1)

    sc_r = jnp.sum(bf(ecl_ref[...]) * bf(wa_r)[:, None, :], axis=2)
    sc_r = sc_r - jnp.max(sc_r, axis=1, keepdims=True)
    a_r = jnp.exp(sc_r)
    a_r = a_r / jnp.sum(a_r, axis=1, keepdims=True)
    agg_r = jnp.sum(ecl_ref[...] * a_r[:, :, None], axis=1)

    ww = wl + wr
    cw = agg_l + agg_r
    pv_ref[...] = jnp.tanh(ww @ Wc1_ref[...] + cw @ Wc2_ref[...]
                           + bc_ref[...])


def _attention(wl, wr, ecl3, ecr3, W_a, b_a, Wc1, Wc2, b_c):
    return pl.pallas_call(
        _attn_body,
        out_shape=jax.ShapeDtypeStruct((B, DIM), jnp.float32),
    )(wl, wr, ecl3, ecr3, W_a, b_a, Wc1, Wc2, b_c)


def _tc_body(pv_in_ref, tab_ref, out_ref, lg_ref, m_ref, s_ref):
    ph = pl.program_id(0)
    t = pl.program_id(1)

    @pl.when((ph == 0) & (t == 0))
    def _init():
        m_ref[...] = jnp.full((B, 1), -jnp.inf, jnp.float32)
        s_ref[...] = jnp.zeros((B, 1), jnp.float32)

    @pl.when(ph == 0)
    def _phase_stats():
        logits = lax.dot_general(pv_in_ref[...], tab_ref[...],
                                 (((1,), (1,)), ((), ())),
                                 preferred_element_type=jnp.float32)
        col = t * _TW + lax.broadcasted_iota(jnp.int32, (B, _TW), 1)
        lg = jnp.where(col < NUM_CANDS, logits, -jnp.inf)
        lg_ref[:, pl.ds(t * _TW, _TW)] = lg
        tmax = jnp.max(lg, axis=1, keepdims=True)
        m_old = m_ref[...]
        m_new = jnp.maximum(m_old, tmax)
        s_ref[...] = (s_ref[...] * jnp.exp(m_old - m_new)
                      + jnp.sum(jnp.exp(lg - m_new), axis=1, keepdims=True))
        m_ref[...] = m_new

    @pl.when(ph == 1)
    def _phase_write():
        lg = lg_ref[:, pl.ds(t * _TW, _TW)]
        out_ref[...] = jnp.exp(lg - m_ref[...]) / s_ref[...]


def _tc_softmax(pv, cand_tab):
    return pl.pallas_call(
        _tc_body,
        grid=(2, _NT),
        in_specs=[
            pl.BlockSpec((B, DIM), lambda ph, t: (0, 0)),
            pl.BlockSpec((_TW, DIM),
                         lambda ph, t: (jnp.where(ph == 0, t, 0), 0)),
        ],
        out_specs=pl.BlockSpec((B, _TW),
                               lambda ph, t: (0, jnp.where(ph == 0, 0, t))),
        out_shape=jax.ShapeDtypeStruct((B, NUM_CANDS), jnp.float32),
        scratch_shapes=[
            pltpu.VMEM((B, _NT * _TW), jnp.float32),
            pltpu.VMEM((B, 1), jnp.float32),
            pltpu.VMEM((B, 1), jnp.float32),
        ],
        compiler_params=pltpu.CompilerParams(
            dimension_semantics=("arbitrary", "arbitrary")),
    )(pv, cand_tab)


_ROWS_PER_W = B // NW
_BLK = 800
_NBLK = NUM_CANDS // _BLK
_VPB = _BLK // 16


def _sc_topk_body(y_hbm, out_hbm, buf, bmax, idxv):
    wid = lax.axis_index("s") * SC_CORES + lax.axis_index("c")
    neg = jnp.full((16,), -1.0, jnp.float32)
    lane = lax.iota(jnp.int32, 16)
    lane0 = lane == 0

    for r in range(_ROWS_PER_W):
        row = wid * _ROWS_PER_W + r
        pltpu.sync_copy(y_hbm.at[row], buf)
        idxv[...] = jnp.zeros((16,), jnp.int32)
        for i in range(8):
            bmax[pl.ds(i * 16, 16)] = neg

        def _bm(j, carry):
            accs = [neg, neg, neg, neg]
            for i in range(_VPB):
                accs[i % 4] = jnp.maximum(
                    accs[i % 4], buf[pl.ds(j * _BLK + i * 16, 16)])
            mx = jnp.max(jnp.maximum(jnp.maximum(accs[0], accs[1]),
                                     jnp.maximum(accs[2], accs[3])))
            plsc.store_scatter(bmax, [jnp.broadcast_to(j, (16,))],
                               jnp.broadcast_to(mx, (16,)), mask=lane0)
            return carry
        lax.fori_loop(0, _NBLK, _bm, 0)

        def _sel(k, carry):
            acc = neg
            for i in range(8):
                acc = jnp.maximum(acc, bmax[pl.ds(i * 16, 16)])
            mx = jnp.max(acc)
            mxs = jnp.broadcast_to(mx, (16,))

            jbv = jnp.full((16,), _NBLK + 16, jnp.int32)
            for i in range(8):
                v = bmax[pl.ds(i * 16, 16)]
                jbv = jnp.minimum(jbv, jnp.where(v == mxs, lane + i * 16,
                                                 _NBLK + 16))
            jb = jnp.min(jbv)
            off = jb * _BLK

            fidv = jnp.full((16,), NUM_CANDS + 16, jnp.int32)
            for i in range(_VPB):
                v = buf[pl.ds(off + i * 16, 16)]
                fidv = jnp.minimum(fidv,
                                   jnp.where(v == mxs,
                                             lane + (off + i * 16),
                                             jnp.int32(NUM_CANDS + 16)))
            fid = jnp.min(fidv)

            plsc.store_scatter(idxv, [jnp.broadcast_to(k, (16,))],
                               jnp.broadcast_to(fid, (16,)), mask=lane0)
            plsc.store_scatter(buf, [jnp.broadcast_to(fid, (16,))],
                               neg, mask=lane0)

            naccs = [neg, neg, neg, neg]
            for i in range(_VPB):
                naccs[i % 4] = jnp.maximum(
                    naccs[i % 4], buf[pl.ds(off + i * 16, 16)])
            nbm = jnp.max(jnp.maximum(jnp.maximum(naccs[0], naccs[1]),
                                      jnp.maximum(naccs[2], naccs[3])))
            plsc.store_scatter(bmax, [jnp.broadcast_to(jb, (16,))],
                               jnp.broadcast_to(nbm, (16,)), mask=lane0)
            return carry
        lax.fori_loop(0, TOP_K, _sel, 0)

        pltpu.sync_copy(idxv, out_hbm.at[row])


@functools.lru_cache(maxsize=None)
def _sc_topk_kernel():
    return pl.kernel(
        _sc_topk_body,
        mesh=_vmesh(),
        out_type=jax.ShapeDtypeStruct((B, 16), jnp.int32),
        scratch_types=[
            pltpu.VMEM((NUM_CANDS,), jnp.float32),
            pltpu.VMEM((128,), jnp.float32),
            pltpu.VMEM((16,), jnp.int32),
        ],
        compiler_params=pltpu.CompilerParams(needs_layout_passes=False),
    )


def kernel(entities, candidates_l, candidates_r, word_embeds,
           candidate_embeds, W_a, b_a, W_c, b_c):
    widx = jnp.concatenate([entities[:, 2], entities[:, 3]])
    cl = candidates_l.reshape(-1)
    cr = candidates_r.reshape(-1)

    wrows, ecl, ecr = _sc_gather_kernel()(widx, cl, cr, word_embeds,
                                          candidate_embeds)
    wl, wr = wrows[:B], wrows[B:]
    ecl3 = ecl.reshape(B, C, DIM)
    ecr3 = ecr.reshape(B, C, DIM)

    pv = _attention(wl, wr, ecl3, ecr3,
                    W_a, b_a, W_c[:DIM], W_c[DIM:], b_c)
    y_pred = _tc_softmax(pv, candidate_embeds)
    top16 = _sc_topk_kernel()(y_pred)
    return y_pred, top16[:, :TOP_K]
